# Initial kernel scaffold; baseline (speedup 1.0000x reference)
import jax
import jax.numpy as jnp
from jax.experimental import pallas as pl


def kernel(x, W0, b0, W1, b1):
    raise NotImplementedError("write your pallas kernel here")


# speedup vs baseline: 1.9346x; 1.9346x over previous
import jax
import jax.numpy as jnp
from jax.experimental import pallas as pl

C = 192
L = 20
H = 224
W = 224
P = H * W
BLK = 3584


def _chain_body(x_ref, w_ref, b_ref, o_ref):
    acc = x_ref[...]
    for l in range(L):
        acc = jnp.dot(w_ref[l], acc, preferred_element_type=jnp.float32)
        acc = acc + b_ref[l][:, None]
    o_ref[...] = acc


def kernel(x, W0, b0, W1, b1):
    x2 = x.reshape(C, P)
    out = pl.pallas_call(
        _chain_body,
        grid=(P // BLK,),
        in_specs=[
            pl.BlockSpec((C, BLK), lambda i: (0, i)),
            pl.BlockSpec((L, C, C), lambda i: (0, 0, 0)),
            pl.BlockSpec((L, C), lambda i: (0, 0)),
        ],
        out_specs=pl.BlockSpec((C, BLK), lambda i: (0, i)),
        out_shape=jax.ShapeDtypeStruct((C, P), jnp.float32),
    )(x2, W0, b0)
    return out.reshape(1, C, H, W)
